# Initial kernel scaffold; baseline (speedup 1.0000x reference)
import jax
import jax.numpy as jnp
from jax.experimental import pallas as pl


def kernel(x, edge_index, theta_direct, theta_heat1, theta_hidden, theta_heat2, t):
    raise NotImplementedError("write your pallas kernel here")


# speedup vs baseline: 5.1184x; 5.1184x over previous
import functools
import math

import jax
import jax.numpy as jnp
import numpy as np
from jax import lax
from jax.experimental import pallas as pl
from jax.experimental.pallas import tpu as pltpu
from jax.experimental.pallas import tpu_sc as plsc

N = 10000
E = 320000
D = 128
K = 10

NC = 2
NS = 16
NW = NC * NS
EPW = E // NW
CH = 80
NCHUNK = EPW // CH
RPS = 640
RPS_LAST = N - RPS * (NS - 1)

_TCR = 1000
_GRID = N // _TCR


ZR = 80


def _seg_body(v_hbm, col_hbm, row_hbm, zero_hbm, p_hbm, colv, rowv, gbuf, zbuf,
              acc, sem):
    c = lax.axis_index("c")
    s = lax.axis_index("s")
    wid = c * NS + s
    rbase = pl.multiple_of(s * RPS, 8)
    nhop = jnp.where(s == NS - 1, RPS_LAST // ZR, RPS // ZR)

    pltpu.sync_copy(zero_hbm, zbuf)

    def zhop(j, carry):
        b = pl.multiple_of(rbase + j * ZR, 8)
        pltpu.sync_copy(zbuf, acc.at[pl.ds(b, ZR)])
        return carry

    lax.fori_loop(0, nhop, zhop, 0)
    plsc.subcore_barrier()
    ebase = wid * EPW

    def chunk(j, carry):
        base = pl.multiple_of(ebase + j * CH, 8)
        pltpu.sync_copy(col_hbm.at[pl.ds(base, CH)], colv)
        pltpu.sync_copy(row_hbm.at[pl.ds(base, CH)], rowv)
        pltpu.async_copy(v_hbm.at[colv], gbuf, sem).wait()
        pltpu.sync_copy(gbuf, acc.at[rowv], add=True)
        return carry

    lax.fori_loop(0, NCHUNK, chunk, 0)
    plsc.subcore_barrier()

    def ohop(j, carry):
        b = pl.multiple_of(rbase + j * ZR, 8)
        pltpu.sync_copy(acc.at[pl.ds(b, ZR)], gbuf)
        pltpu.sync_copy(gbuf, p_hbm.at[pl.ds(c * N + b, ZR)])
        return carry

    lax.fori_loop(0, nhop, ohop, 0)


_seg = pl.kernel(
    _seg_body,
    out_type=jax.ShapeDtypeStruct((NC * N, D), jnp.float32),
    mesh=plsc.VectorSubcoreMesh(core_axis_name="c", subcore_axis_name="s"),
    scratch_types=[
        pltpu.VMEM((CH,), jnp.int32),
        pltpu.VMEM((CH,), jnp.int32),
        pltpu.VMEM((CH, D), jnp.float32),
        pltpu.VMEM((ZR, D), jnp.float32),
        pltpu.VMEM_SHARED((N, D), jnp.float32),
        pltpu.SemaphoreType.DMA,
    ],
)


def _prep_body(c0_ref, p_ref, x_ref, dinv_ref, g_ref, out_ref):
    s = p_ref[0] + p_ref[1]
    dinv = jnp.where(s > 0, lax.rsqrt(jnp.maximum(s, 1e-12)), 0.0)
    x = x_ref[...]
    dinv_ref[...] = dinv
    g_ref[...] = dinv * x
    out_ref[...] = c0_ref[0, 0] * x


_prep = pl.pallas_call(
    _prep_body,
    grid=(_GRID,),
    in_specs=[
        pl.BlockSpec(memory_space=pltpu.SMEM),
        pl.BlockSpec((2, _TCR, D), lambda i: (0, i, 0)),
        pl.BlockSpec((_TCR, D), lambda i: (i, 0)),
    ],
    out_specs=[
        pl.BlockSpec((_TCR, D), lambda i: (i, 0)),
        pl.BlockSpec((_TCR, D), lambda i: (i, 0)),
        pl.BlockSpec((_TCR, D), lambda i: (i, 0)),
    ],
    out_shape=[jax.ShapeDtypeStruct((N, D), jnp.float32)] * 3,
)


def _combine_body(ck_ref, p_ref, tm2_ref, dinv_ref, outin_ref,
                  t_ref, g_ref, outnew_ref, *, first):
    s = p_ref[0] + p_ref[1]
    dinv = dinv_ref[...]
    if first:
        t = -dinv * s
    else:
        t = -2.0 * (dinv * s) - tm2_ref[...]
    t_ref[...] = t
    g_ref[...] = dinv * t
    outnew_ref[...] = outin_ref[...] + ck_ref[0, 0] * t


def _make_combine(first):
    return pl.pallas_call(
        functools.partial(_combine_body, first=first),
        grid=(_GRID,),
        in_specs=[
            pl.BlockSpec(memory_space=pltpu.SMEM),
            pl.BlockSpec((2, _TCR, D), lambda i: (0, i, 0)),
            pl.BlockSpec((_TCR, D), lambda i: (i, 0)),
            pl.BlockSpec((_TCR, D), lambda i: (i, 0)),
            pl.BlockSpec((_TCR, D), lambda i: (i, 0)),
        ],
        out_specs=[
            pl.BlockSpec((_TCR, D), lambda i: (i, 0)),
            pl.BlockSpec((_TCR, D), lambda i: (i, 0)),
            pl.BlockSpec((_TCR, D), lambda i: (i, 0)),
        ],
        out_shape=[jax.ShapeDtypeStruct((N, D), jnp.float32)] * 3,
    )


_combine_first = _make_combine(True)
_combine_rest = _make_combine(False)


def _mid_body(c0_ref, x_ref, xh_ref, td_ref, th1_ref, dinv_ref,
              hid_ref, g_ref, out_ref):
    h = jnp.dot(x_ref[...], td_ref[...], preferred_element_type=jnp.float32)
    h += jnp.dot(xh_ref[...], th1_ref[...], preferred_element_type=jnp.float32)
    h = jnp.maximum(h, 0.0)
    hid_ref[...] = h
    g_ref[...] = dinv_ref[...] * h
    out_ref[...] = c0_ref[0, 0] * h


_mid = pl.pallas_call(
    _mid_body,
    grid=(_GRID,),
    in_specs=[
        pl.BlockSpec(memory_space=pltpu.SMEM),
        pl.BlockSpec((_TCR, D), lambda i: (i, 0)),
        pl.BlockSpec((_TCR, D), lambda i: (i, 0)),
        pl.BlockSpec((D, D), lambda i: (0, 0)),
        pl.BlockSpec((D, D), lambda i: (0, 0)),
        pl.BlockSpec((_TCR, D), lambda i: (i, 0)),
    ],
    out_specs=[
        pl.BlockSpec((_TCR, D), lambda i: (i, 0)),
        pl.BlockSpec((_TCR, D), lambda i: (i, 0)),
        pl.BlockSpec((_TCR, D), lambda i: (i, 0)),
    ],
    out_shape=[jax.ShapeDtypeStruct((N, D), jnp.float32)] * 3,
)


def _final_body(h_ref, hh_ref, th_ref, th2_ref, o_ref):
    z = jnp.dot(h_ref[...], th_ref[...], preferred_element_type=jnp.float32)
    z += jnp.dot(hh_ref[...], th2_ref[...], preferred_element_type=jnp.float32)
    m = jnp.max(z, axis=1, keepdims=True)
    lse = m + jnp.log(jnp.sum(jnp.exp(z - m), axis=1, keepdims=True))
    o_ref[...] = z - lse


_final = pl.pallas_call(
    _final_body,
    grid=(_GRID,),
    in_specs=[
        pl.BlockSpec((_TCR, D), lambda i: (i, 0)),
        pl.BlockSpec((_TCR, D), lambda i: (i, 0)),
        pl.BlockSpec((D, D), lambda i: (0, 0)),
        pl.BlockSpec((D, D), lambda i: (0, 0)),
    ],
    out_specs=pl.BlockSpec((_TCR, D), lambda i: (i, 0)),
    out_shape=jax.ShapeDtypeStruct((N, D), jnp.float32),
)


_M30 = np.arange(30, dtype=np.float32)
_LGAMMA = np.array(
    [[math.lgamma(m + 1.0) + math.lgamma(m + k + 1.0) for m in range(30)]
     for k in range(K)], dtype=np.float32)


def _coeffs(t):
    lt = jnp.log(t / 2.0)
    cs = []
    for k in range(K):
        ik = jnp.sum(jnp.exp((2.0 * _M30 + k) * lt - _LGAMMA[k]))
        ck = ik if k == 0 else 2.0 * ((-1.0) ** k) * ik
        cs.append(jnp.reshape(ck.astype(jnp.float32), (1, 1)))
    return cs


def _heat_sweep(g0, out_acc, x0, col, row, zeros, dinv, cs):
    g = g0
    tm2 = x0
    tm1 = None
    for k in range(1, K):
        p = _seg(g, col, row, zeros).reshape(NC, N, D)
        comb = _combine_first if k == 1 else _combine_rest
        tk, g, out_acc = comb(cs[k], p, tm2, dinv, out_acc)
        tm2, tm1 = (x0, tk) if k == 1 else (tm1, tk)
    return out_acc


def kernel(x, edge_index, theta_direct, theta_heat1, theta_hidden,
           theta_heat2, t):
    row = edge_index[0]
    col = edge_index[1]
    zeros = jnp.zeros((ZR, D), jnp.float32)
    ones = jnp.ones((N, D), jnp.float32)
    cs = _coeffs(t)

    pdeg = _seg(ones, col, row, zeros).reshape(NC, N, D)
    dinv, g0, out1 = _prep(cs[0], pdeg, x)
    x_heat = _heat_sweep(g0, out1, x, col, row, zeros, dinv, cs)

    hidden, gh0, out2 = _mid(cs[0], x, x_heat, theta_direct, theta_heat1,
                             dinv)
    hidden_heat = _heat_sweep(gh0, out2, hidden, col, row, zeros, dinv, cs)

    return _final(hidden, hidden_heat, theta_hidden, theta_heat2)
